# baseline (speedup 1.0000x reference)
import jax
import jax.numpy as jnp
from jax import lax
from jax.experimental import pallas as pl
from jax.experimental.pallas import tpu as pltpu

_N = 256
_D = 128
_MARGIN = 1.0
_BIG = 1e9
_BLK = 8
_NBLK = _N // _BLK
_W = 64
_PAD = _N + _W * 2


def _triplet_kernel(emb_ref, trow_ref, tcol_ref, kx_ref, loss_ref,
                    ratio_ref, b_s, as3):
    E = emb_ref[:]
    t_row = trow_ref[:]
    t_col = tcol_ref[:]

    G = lax.dot_general(E, E, (((1,), (1,)), ((), ())),
                        preferred_element_type=jnp.float32)
    EE = E * E
    sq_col = jnp.sum(EE, axis=1, keepdims=True)
    ones_d = jnp.ones((1, _D), jnp.float32)
    sq_row = lax.dot_general(ones_d, EE, (((1,), (1,)), ((), ())),
                             preferred_element_type=jnp.float32)
    Dm = sq_col + sq_row - 2.0 * G

    same = t_col == t_row
    row_i = lax.broadcasted_iota(jnp.int32, (_N, _N), 0)
    col_i = lax.broadcasted_iota(jnp.int32, (_N, _N), 1)
    apf = jnp.where(same & (row_i < col_i), 1.0, 0.0)
    negf = jnp.where(same, 0.0, 1.0)

    b_s[...] = jnp.where(negf > 0.5, Dm, _BIG)

    padv = jnp.full((_PAD - _N, _BLK), -_BIG, jnp.float32)
    rowj = lax.broadcasted_iota(jnp.int32, (_N, _BLK), 0)
    cola = lax.broadcasted_iota(jnp.int32, (_N, _BLK), 1)
    for b in range(_NBLK):
        i0 = b * _BLK
        dcol = Dm[:, i0:i0 + _BLK]
        tcb = t_row[:, i0:i0 + _BLK]
        ap_cb = (t_col == tcb) & (rowj > cola + i0)
        as3[b, 0:_N, :] = jnp.where(ap_cb, dcol + _MARGIN, -_BIG)
        as3[b, _N:_PAD, :] = padv

    def body(b, carry):
        tot_acc, vio_acc = carry
        i0 = b * _BLK
        Bm = b_s[pl.ds(i0, _BLK), :]

        def win(jw, tacc, vacc):
            Ap = as3[b, pl.ds(jw, _W), :]
            T = Ap[:, :, None] - Bm[None, :, :]
            tacc = tacc + jnp.maximum(T, 0.0)
            vacc = vacc + jnp.where(T > 0.0, 1.0, 0.0)
            return tacc, vacc

        tot_acc, vio_acc = win(i0, tot_acc, vio_acc)

        def fb(q, c):
            return win(i0 + _W + q * _W, c[0], c[1])

        kx = kx_ref[0, b]
        tot_acc, vio_acc = lax.fori_loop(0, kx, fb, (tot_acc, vio_acc))
        return tot_acc, vio_acc

    zeros = jnp.zeros((_W, _BLK, _N), jnp.float32)
    tot_acc, vio_acc = lax.fori_loop(0, _NBLK, body, (zeros, zeros))

    total = jnp.sum(tot_acc)
    viol = jnp.sum(vio_acc)

    ones_n = jnp.ones((1, _N), jnp.float32)
    p_row = lax.dot_general(ones_n, apf, (((1,), (1,)), ((), ())),
                            preferred_element_type=jnp.float32)
    m_row = lax.dot_general(ones_n, negf, (((1,), (1,)), ((), ())),
                            preferred_element_type=jnp.float32)
    count = jnp.sum(p_row * m_row)

    has = count > 0.5
    loss_sum = jnp.where(has, total, jnp.float32(1.0))
    ratio = jnp.where(has, viol / jnp.maximum(count, 1.0),
                      jnp.float32(1.0))
    loss_ref[...] = jnp.broadcast_to(loss_sum, (1, 1))
    ratio_ref[...] = jnp.broadcast_to(ratio, (1, 1))


def kernel(embeddings, target):
    t32 = target.astype(jnp.int32)
    perm = jnp.argsort(t32)
    ts = t32[perm]
    es = embeddings.astype(jnp.float32)[perm]
    i0s = _BLK * jnp.arange(_NBLK, dtype=jnp.int32)
    thr = ts[i0s + _BLK - 1]
    ends = jnp.sum(ts[None, :] <= thr[:, None], axis=1).astype(jnp.int32)
    kx = jnp.maximum(0, -((ends - i0s - _W) // -_W)).astype(jnp.int32)
    loss, ratio = pl.pallas_call(
        _triplet_kernel,
        out_shape=(jax.ShapeDtypeStruct((1, 1), jnp.float32),
                   jax.ShapeDtypeStruct((1, 1), jnp.float32)),
        in_specs=[pl.BlockSpec(memory_space=pltpu.VMEM),
                  pl.BlockSpec(memory_space=pltpu.VMEM),
                  pl.BlockSpec(memory_space=pltpu.VMEM),
                  pl.BlockSpec(memory_space=pltpu.SMEM)],
        scratch_shapes=[pltpu.VMEM((_N, _N), jnp.float32),
                        pltpu.VMEM((_NBLK, _PAD, _BLK), jnp.float32)],
    )(es, ts.reshape(1, _N), ts.reshape(_N, 1), kx.reshape(1, _NBLK))
    return (loss[0, 0], ratio[0, 0])


# speedup vs baseline: 1.3278x; 1.3278x over previous
import jax
import jax.numpy as jnp
from jax import lax
from jax.experimental import pallas as pl
from jax.experimental.pallas import tpu as pltpu

_N = 256
_D = 128
_MARGIN = 1.0
_BIG = 1e9
_BLK = 8
_NBLK = _N // _BLK
_W = 64
_PAD = _N + _W * 2


def _triplet_kernel(emb_ref, trow_ref, tcol_ref, kx_ref, loss_ref,
                    ratio_ref, b_s, as3):
    E = emb_ref[:]
    t_row = trow_ref[:]
    t_col = tcol_ref[:]

    G = lax.dot_general(E, E, (((1,), (1,)), ((), ())),
                        preferred_element_type=jnp.float32)
    EE = E * E
    sq_col = jnp.sum(EE, axis=1, keepdims=True)
    ones_d = jnp.ones((1, _D), jnp.float32)
    sq_row = lax.dot_general(ones_d, EE, (((1,), (1,)), ((), ())),
                             preferred_element_type=jnp.float32)
    Dm = sq_col + sq_row - 2.0 * G

    same = t_col == t_row
    row_i = lax.broadcasted_iota(jnp.int32, (_N, _N), 0)
    col_i = lax.broadcasted_iota(jnp.int32, (_N, _N), 1)
    apf = jnp.where(same & (row_i < col_i), 1.0, 0.0)
    negf = jnp.where(same, 0.0, 1.0)

    b_s[...] = jnp.where(negf > 0.5, Dm, _BIG)

    padv = jnp.full((_PAD - _N, _BLK), -_BIG, jnp.float32)
    rowj = lax.broadcasted_iota(jnp.int32, (_N, _BLK), 0)
    cola = lax.broadcasted_iota(jnp.int32, (_N, _BLK), 1)
    for b in range(_NBLK):
        i0 = b * _BLK
        dcol = Dm[:, i0:i0 + _BLK]
        tcb = t_row[:, i0:i0 + _BLK]
        ap_cb = (t_col == tcb) & (rowj > cola + i0)
        as3[b, 0:_N, :] = jnp.where(ap_cb, dcol + _MARGIN, -_BIG)
        as3[b, _N:_PAD, :] = padv

    def body(b, carry):
        tot_acc, vio_acc = carry
        i0 = b * _BLK
        Bm = b_s[pl.ds(i0, _BLK), :]

        def tree0(x):
            n = x.shape[0]
            while n > 1:
                h = n // 2
                x = x[0:h] + x[h:n]
                n = h
            return x[0]

        def win(jw, tacc, vacc):
            Ap = as3[b, pl.ds(jw, _W), :]
            T = Ap[:, :, None] - Bm[None, :, :]
            tacc = tacc + tree0(jnp.maximum(T, 0.0))
            vacc = vacc + tree0(jnp.where(T > 0.0, 1.0, 0.0))
            return tacc, vacc

        tot_acc, vio_acc = win(i0, tot_acc, vio_acc)

        def fb(q, c):
            return win(i0 + _W + q * _W, c[0], c[1])

        kx = kx_ref[0, b]
        tot_acc, vio_acc = lax.fori_loop(0, kx, fb, (tot_acc, vio_acc))
        return tot_acc, vio_acc

    zeros = jnp.zeros((_BLK, _N), jnp.float32)
    tot_acc, vio_acc = lax.fori_loop(0, _NBLK, body, (zeros, zeros))

    total = jnp.sum(tot_acc)
    viol = jnp.sum(vio_acc)

    ones_n = jnp.ones((1, _N), jnp.float32)
    p_row = lax.dot_general(ones_n, apf, (((1,), (1,)), ((), ())),
                            preferred_element_type=jnp.float32)
    m_row = lax.dot_general(ones_n, negf, (((1,), (1,)), ((), ())),
                            preferred_element_type=jnp.float32)
    count = jnp.sum(p_row * m_row)

    has = count > 0.5
    loss_sum = jnp.where(has, total, jnp.float32(1.0))
    ratio = jnp.where(has, viol / jnp.maximum(count, 1.0),
                      jnp.float32(1.0))
    loss_ref[...] = jnp.broadcast_to(loss_sum, (1, 1))
    ratio_ref[...] = jnp.broadcast_to(ratio, (1, 1))


def kernel(embeddings, target):
    t32 = target.astype(jnp.int32)
    perm = jnp.argsort(t32)
    ts = t32[perm]
    es = embeddings.astype(jnp.float32)[perm]
    i0s = _BLK * jnp.arange(_NBLK, dtype=jnp.int32)
    thr = ts[i0s + _BLK - 1]
    ends = jnp.sum(ts[None, :] <= thr[:, None], axis=1).astype(jnp.int32)
    kx = jnp.maximum(0, -((ends - i0s - _W) // -_W)).astype(jnp.int32)
    loss, ratio = pl.pallas_call(
        _triplet_kernel,
        out_shape=(jax.ShapeDtypeStruct((1, 1), jnp.float32),
                   jax.ShapeDtypeStruct((1, 1), jnp.float32)),
        in_specs=[pl.BlockSpec(memory_space=pltpu.VMEM),
                  pl.BlockSpec(memory_space=pltpu.VMEM),
                  pl.BlockSpec(memory_space=pltpu.VMEM),
                  pl.BlockSpec(memory_space=pltpu.SMEM)],
        scratch_shapes=[pltpu.VMEM((_N, _N), jnp.float32),
                        pltpu.VMEM((_NBLK, _PAD, _BLK), jnp.float32)],
    )(es, ts.reshape(1, _N), ts.reshape(_N, 1), kx.reshape(1, _NBLK))
    return (loss[0, 0], ratio[0, 0])


# speedup vs baseline: 1.4212x; 1.0703x over previous
import jax
import jax.numpy as jnp
from jax import lax
from jax.experimental import pallas as pl
from jax.experimental.pallas import tpu as pltpu

_N = 256
_D = 128
_MARGIN = 1.0
_BIG = 1e9
_BLK = 16
_NBLK = _N // _BLK


def _triplet_kernel(emb_ref, trow_ref, tcol_ref, loss_ref, ratio_ref,
                    a_s, b_s):
    E = emb_ref[:]
    t_row = trow_ref[:]
    t_col = tcol_ref[:]

    G = lax.dot_general(E, E, (((1,), (1,)), ((), ())),
                        preferred_element_type=jnp.float32)
    EE = E * E
    sq_col = jnp.sum(EE, axis=1, keepdims=True)
    ones_d = jnp.ones((1, _D), jnp.float32)
    sq_row = lax.dot_general(ones_d, EE, (((1,), (1,)), ((), ())),
                             preferred_element_type=jnp.float32)
    Dm = sq_col + sq_row - 2.0 * G

    same = t_col == t_row
    row_i = lax.broadcasted_iota(jnp.int32, (_N, _N), 0)
    col_i = lax.broadcasted_iota(jnp.int32, (_N, _N), 1)
    apf = jnp.where(same & (row_i < col_i), 1.0, 0.0)
    negf = jnp.where(same, 0.0, 1.0)

    a_s[...] = jnp.where(apf > 0.5, Dm + _MARGIN, -_BIG)
    b_s[...] = jnp.where(negf > 0.5, Dm, _BIG)

    def body(bi, carry):
        tot_acc, viol_acc = carry
        i0 = bi * _BLK
        A = a_s[pl.ds(i0, _BLK), :]
        B = b_s[pl.ds(i0, _BLK), :]
        T = A[:, :, None] - B[:, None, :]
        tot_acc = tot_acc + jnp.sum(jnp.maximum(T, 0.0), axis=1)
        viol_acc = viol_acc + jnp.sum(jnp.where(T > 0.0, 1.0, 0.0),
                                      axis=1)
        return tot_acc, viol_acc

    zeros = jnp.zeros((_BLK, _N), jnp.float32)
    tot_acc, viol_acc = lax.fori_loop(0, _NBLK, body, (zeros, zeros))

    total = jnp.sum(tot_acc)
    viol = jnp.sum(viol_acc)

    ones_n = jnp.ones((1, _N), jnp.float32)
    p_row = lax.dot_general(ones_n, apf, (((1,), (1,)), ((), ())),
                            preferred_element_type=jnp.float32)
    m_row = lax.dot_general(ones_n, negf, (((1,), (1,)), ((), ())),
                            preferred_element_type=jnp.float32)
    count = jnp.sum(p_row * m_row)

    has = count > 0.5
    loss_sum = jnp.where(has, total, jnp.float32(1.0))
    ratio = jnp.where(has, viol / jnp.maximum(count, 1.0),
                      jnp.float32(1.0))
    loss_ref[...] = jnp.broadcast_to(loss_sum, (1, 1))
    ratio_ref[...] = jnp.broadcast_to(ratio, (1, 1))


def kernel(embeddings, target):
    t32 = target.astype(jnp.int32)
    t_row = t32.reshape(1, _N)
    t_col = t32.reshape(_N, 1)
    loss, ratio = pl.pallas_call(
        _triplet_kernel,
        out_shape=(jax.ShapeDtypeStruct((1, 1), jnp.float32),
                   jax.ShapeDtypeStruct((1, 1), jnp.float32)),
        scratch_shapes=[pltpu.VMEM((_N, _N), jnp.float32),
                        pltpu.VMEM((_N, _N), jnp.float32)],
    )(embeddings.astype(jnp.float32), t_row, t_col)
    return (loss[0, 0], ratio[0, 0])


# speedup vs baseline: 1.4573x; 1.0255x over previous
import jax
import jax.numpy as jnp
from jax import lax
from jax.experimental import pallas as pl
from jax.experimental.pallas import tpu as pltpu

_N = 256
_D = 128
_MARGIN = 1.0
_BIG = 1e9
_BLK = 32
_NBLK = _N // _BLK


def _triplet_kernel(emb_ref, trow_ref, tcol_ref, loss_ref, ratio_ref,
                    a_s, b_s):
    E = emb_ref[:]
    t_row = trow_ref[:]
    t_col = tcol_ref[:]

    G = lax.dot_general(E, E, (((1,), (1,)), ((), ())),
                        preferred_element_type=jnp.float32)
    EE = E * E
    sq_col = jnp.sum(EE, axis=1, keepdims=True)
    ones_d = jnp.ones((1, _D), jnp.float32)
    sq_row = lax.dot_general(ones_d, EE, (((1,), (1,)), ((), ())),
                             preferred_element_type=jnp.float32)
    Dm = sq_col + sq_row - 2.0 * G

    same = t_col == t_row
    row_i = lax.broadcasted_iota(jnp.int32, (_N, _N), 0)
    col_i = lax.broadcasted_iota(jnp.int32, (_N, _N), 1)
    apf = jnp.where(same & (row_i < col_i), 1.0, 0.0)
    negf = jnp.where(same, 0.0, 1.0)

    a_s[...] = jnp.where(apf > 0.5, Dm + _MARGIN, -_BIG)
    b_s[...] = jnp.where(negf > 0.5, Dm, _BIG)

    def body(bi, carry):
        tot_acc, viol_acc = carry
        i0 = bi * _BLK
        A = a_s[pl.ds(i0, _BLK), :]
        B = b_s[pl.ds(i0, _BLK), :]
        T = A[:, :, None] - B[:, None, :]
        tot_acc = tot_acc + jnp.sum(jnp.maximum(T, 0.0), axis=1)
        viol_acc = viol_acc + jnp.sum(jnp.where(T > 0.0, 1.0, 0.0),
                                      axis=1)
        return tot_acc, viol_acc

    zeros = jnp.zeros((_BLK, _N), jnp.float32)
    tot_acc, viol_acc = lax.fori_loop(0, _NBLK, body, (zeros, zeros))

    total = jnp.sum(tot_acc)
    viol = jnp.sum(viol_acc)

    ones_n = jnp.ones((1, _N), jnp.float32)
    p_row = lax.dot_general(ones_n, apf, (((1,), (1,)), ((), ())),
                            preferred_element_type=jnp.float32)
    m_row = lax.dot_general(ones_n, negf, (((1,), (1,)), ((), ())),
                            preferred_element_type=jnp.float32)
    count = jnp.sum(p_row * m_row)

    has = count > 0.5
    loss_sum = jnp.where(has, total, jnp.float32(1.0))
    ratio = jnp.where(has, viol / jnp.maximum(count, 1.0),
                      jnp.float32(1.0))
    loss_ref[...] = jnp.broadcast_to(loss_sum, (1, 1))
    ratio_ref[...] = jnp.broadcast_to(ratio, (1, 1))


def kernel(embeddings, target):
    t32 = target.astype(jnp.int32)
    t_row = t32.reshape(1, _N)
    t_col = t32.reshape(_N, 1)
    loss, ratio = pl.pallas_call(
        _triplet_kernel,
        out_shape=(jax.ShapeDtypeStruct((1, 1), jnp.float32),
                   jax.ShapeDtypeStruct((1, 1), jnp.float32)),
        scratch_shapes=[pltpu.VMEM((_N, _N), jnp.float32),
                        pltpu.VMEM((_N, _N), jnp.float32)],
    )(embeddings.astype(jnp.float32), t_row, t_col)
    return (loss[0, 0], ratio[0, 0])


# speedup vs baseline: 1.4631x; 1.0039x over previous
import jax
import jax.numpy as jnp
from jax import lax
from jax.experimental import pallas as pl
from jax.experimental.pallas import tpu as pltpu

_N = 256
_D = 128
_MARGIN = 1.0
_BIG = 1e9
_BLK = 64
_NBLK = _N // _BLK


def _triplet_kernel(emb_ref, trow_ref, tcol_ref, loss_ref, ratio_ref,
                    a_s, b_s):
    E = emb_ref[:]
    t_row = trow_ref[:]
    t_col = tcol_ref[:]

    G = lax.dot_general(E, E, (((1,), (1,)), ((), ())),
                        preferred_element_type=jnp.float32)
    EE = E * E
    sq_col = jnp.sum(EE, axis=1, keepdims=True)
    ones_d = jnp.ones((1, _D), jnp.float32)
    sq_row = lax.dot_general(ones_d, EE, (((1,), (1,)), ((), ())),
                             preferred_element_type=jnp.float32)
    Dm = sq_col + sq_row - 2.0 * G

    same = t_col == t_row
    row_i = lax.broadcasted_iota(jnp.int32, (_N, _N), 0)
    col_i = lax.broadcasted_iota(jnp.int32, (_N, _N), 1)
    apf = jnp.where(same & (row_i < col_i), 1.0, 0.0)
    negf = jnp.where(same, 0.0, 1.0)

    a_s[...] = jnp.where(apf > 0.5, Dm + _MARGIN, -_BIG)
    b_s[...] = jnp.where(negf > 0.5, Dm, _BIG)

    def body(bi, carry):
        tot_acc, viol_acc = carry
        i0 = bi * _BLK
        A = a_s[pl.ds(i0, _BLK), :]
        B = b_s[pl.ds(i0, _BLK), :]
        T = A[:, :, None] - B[:, None, :]
        tot_acc = tot_acc + jnp.sum(jnp.maximum(T, 0.0), axis=1)
        viol_acc = viol_acc + jnp.sum(jnp.where(T > 0.0, 1.0, 0.0),
                                      axis=1)
        return tot_acc, viol_acc

    zeros = jnp.zeros((_BLK, _N), jnp.float32)
    tot_acc, viol_acc = lax.fori_loop(0, _NBLK, body, (zeros, zeros))

    total = jnp.sum(tot_acc)
    viol = jnp.sum(viol_acc)

    ones_n = jnp.ones((1, _N), jnp.float32)
    p_row = lax.dot_general(ones_n, apf, (((1,), (1,)), ((), ())),
                            preferred_element_type=jnp.float32)
    m_row = lax.dot_general(ones_n, negf, (((1,), (1,)), ((), ())),
                            preferred_element_type=jnp.float32)
    count = jnp.sum(p_row * m_row)

    has = count > 0.5
    loss_sum = jnp.where(has, total, jnp.float32(1.0))
    ratio = jnp.where(has, viol / jnp.maximum(count, 1.0),
                      jnp.float32(1.0))
    loss_ref[...] = jnp.broadcast_to(loss_sum, (1, 1))
    ratio_ref[...] = jnp.broadcast_to(ratio, (1, 1))


def kernel(embeddings, target):
    t32 = target.astype(jnp.int32)
    t_row = t32.reshape(1, _N)
    t_col = t32.reshape(_N, 1)
    loss, ratio = pl.pallas_call(
        _triplet_kernel,
        out_shape=(jax.ShapeDtypeStruct((1, 1), jnp.float32),
                   jax.ShapeDtypeStruct((1, 1), jnp.float32)),
        scratch_shapes=[pltpu.VMEM((_N, _N), jnp.float32),
                        pltpu.VMEM((_N, _N), jnp.float32)],
    )(embeddings.astype(jnp.float32), t_row, t_col)
    return (loss[0, 0], ratio[0, 0])
